# baseline (speedup 1.0000x reference)
import jax
import jax.numpy as jnp
from jax.experimental import pallas as pl


def _sqdist(src, dst):
    return (jnp.sum(src ** 2, -1)[:, :, None] + jnp.sum(dst ** 2, -1)[:, None, :]
            - 2.0 * jnp.einsum('bnc,bmc->bnm', src, dst))


def _index_points(points, idx):
    return jax.vmap(lambda p, i: p[i])(points, idx)


def _fps(xyz, npoint):
    b, n, _ = xyz.shape
    def step(state, _):
        distance, farthest = state
        centroid = _index_points(xyz, farthest[:, None])
        dist = jnp.sum((xyz - centroid) ** 2, -1)
        distance = jnp.minimum(distance, dist)
        new_farthest = jnp.argmax(distance, -1).astype(jnp.int32)
        return (distance, new_farthest), farthest
    init = (jnp.full((b, n), 1e10, jnp.float32), jnp.zeros((b,), jnp.int32))
    _, centroids = jax.lax.scan(step, init, None, length=npoint)
    return jnp.transpose(centroids, (1, 0))


def _query_ball(radius, nsample, xyz, new_xyz):
    b, n, _ = xyz.shape
    s = new_xyz.shape[1]
    sqrdists = _sqdist(new_xyz, xyz)
    group_idx = jnp.broadcast_to(jnp.arange(n, dtype=jnp.int32), (b, s, n))
    group_idx = jnp.where(sqrdists > radius ** 2, n, group_idx)
    group_idx = jnp.sort(group_idx, axis=-1)[:, :, :nsample]
    group_first = jnp.broadcast_to(group_idx[:, :, :1], group_idx.shape)
    group_idx = jnp.where(group_idx == n, group_first, group_idx)
    return group_idx


def _apply_mlp(x, layers):
    for W, bvec in layers:
        x = jax.nn.relu(x @ W + bvec)
    return x


def _set_abstraction(xyz, points, npoint, radius, nsample, layers):
    fps_idx = _fps(xyz, npoint)
    new_xyz = _index_points(xyz, fps_idx)
    idx = _query_ball(radius, nsample, xyz, new_xyz)
    grouped_xyz = _index_points(xyz, idx) - new_xyz[:, :, None, :]
    grouped_points = _index_points(points, idx)
    new_points = jnp.concatenate([grouped_xyz, grouped_points], axis=-1)
    new_points = _apply_mlp(new_points, layers)
    new_points = jnp.max(new_points, axis=2)
    return new_xyz, new_points, fps_idx


def _feature_propagation(xyz1, xyz2, points1, points2, layers):
    dists = _sqdist(xyz1, xyz2)
    neg_vals, idx = jax.lax.top_k(-dists, 3)
    d3 = jnp.maximum(-neg_vals, 0.0)
    dist_recip = 1.0 / (d3 + 1e-8)
    norm = jnp.sum(dist_recip, axis=-1, keepdims=True)
    weight = dist_recip / norm
    interpolated = jnp.sum(_index_points(points2, idx) * weight[..., None], axis=2)
    new_points = jnp.concatenate([points1, interpolated], axis=-1)
    return _apply_mlp(new_points, layers)


def _fc_body(x_ref, w1_ref, b1_ref, w2_ref, b2_ref, o_ref):
    x = x_ref[0]
    h = jnp.maximum(jnp.dot(x, w1_ref[...]) + b1_ref[...], 0.0)
    o_ref[0] = jnp.dot(h, w2_ref[...]) + b2_ref[...]


def _fc_head(up2, fc_params):
    (W1, b1), (W2, b2) = fc_params
    B, N, D = up2.shape
    H = W1.shape[1]
    C = W2.shape[1]
    CP = 128
    W2p = jnp.pad(W2, ((0, 0), (0, CP - C)))
    b2p = jnp.pad(b2, (0, CP - C))
    out = pl.pallas_call(
        _fc_body,
        grid=(B,),
        in_specs=[
            pl.BlockSpec((1, N, D), lambda b: (b, 0, 0)),
            pl.BlockSpec((D, H), lambda b: (0, 0)),
            pl.BlockSpec((1, H), lambda b: (0, 0)),
            pl.BlockSpec((H, CP), lambda b: (0, 0)),
            pl.BlockSpec((1, CP), lambda b: (0, 0)),
        ],
        out_specs=pl.BlockSpec((1, N, CP), lambda b: (b, 0, 0)),
        out_shape=jax.ShapeDtypeStruct((B, N, CP), jnp.float32),
    )(up2, W1, b1.reshape(1, H), W2p, b2p.reshape(1, CP))
    return out[..., :C]


def kernel(xyz, params):
    features = xyz
    new_xyz1, f1, _ = _set_abstraction(xyz, features, 512, 0.2, 32, params['sa1'])
    new_xyz2, f2, _ = _set_abstraction(new_xyz1, f1, 128, 0.4, 64, params['sa2'])
    up1 = _feature_propagation(new_xyz1, new_xyz2, f1, f2, params['fp1'])
    up2 = _feature_propagation(xyz, new_xyz1, features, up1, params['fp2'])
    return _fc_head(up2, params['fc'])


# speedup vs baseline: 2.0335x; 2.0335x over previous
import functools

import jax
import jax.numpy as jnp
from jax.experimental import pallas as pl
from jax.experimental.pallas import tpu as pltpu

B, N, NUM_CLASSES = 8, 4096, 13


def _fps_body(x_ref, y_ref, z_ref, ox_ref, oy_ref, oz_ref, dist_ref):
    b, n = x_ref.shape
    s = ox_ref.shape[1]
    x = x_ref[...]
    y = y_ref[...]
    z = z_ref[...]
    lane = jax.lax.broadcasted_iota(jnp.int32, (b, n), 1)
    lane_s = jax.lax.broadcasted_iota(jnp.int32, (b, s), 1)
    dist_ref[...] = jnp.full((b, n), 1e10, jnp.float32)

    def step(i, carry):
        cx, cy, cz = carry
        ox_ref[...] = jnp.where(lane_s == i, cx, ox_ref[...])
        oy_ref[...] = jnp.where(lane_s == i, cy, oy_ref[...])
        oz_ref[...] = jnp.where(lane_s == i, cz, oz_ref[...])
        dx = x - cx
        dy = y - cy
        dz = z - cz
        d = (dx * dx + dy * dy) + dz * dz
        dist = jnp.minimum(dist_ref[...], d)
        dist_ref[...] = dist
        m = jnp.max(dist, axis=1, keepdims=True)
        sel = jnp.min(jnp.where(dist == m, lane, n), axis=1, keepdims=True)
        oh = lane == sel
        nx = jnp.sum(jnp.where(oh, x, 0.0), axis=1, keepdims=True)
        ny = jnp.sum(jnp.where(oh, y, 0.0), axis=1, keepdims=True)
        nz = jnp.sum(jnp.where(oh, z, 0.0), axis=1, keepdims=True)
        return nx, ny, nz

    jax.lax.fori_loop(0, s, step, (x[:, 0:1], y[:, 0:1], z[:, 0:1]))


def _fps_coords(xyz, npoint):
    b, n, _ = xyz.shape
    x = xyz[..., 0]
    y = xyz[..., 1]
    z = xyz[..., 2]
    shp = jax.ShapeDtypeStruct((b, npoint), jnp.float32)
    ox, oy, oz = pl.pallas_call(
        _fps_body,
        in_specs=[pl.BlockSpec((b, n), lambda: (0, 0))] * 3,
        out_specs=[pl.BlockSpec((b, npoint), lambda: (0, 0))] * 3,
        out_shape=[shp, shp, shp],
        scratch_shapes=[pltpu.VMEM((b, n), jnp.float32)],
    )(x, y, z)
    return jnp.stack([ox, oy, oz], axis=-1)


def _sqdist_jax(src, dst):
    return (jnp.sum(src ** 2, -1)[:, :, None] + jnp.sum(dst ** 2, -1)[:, None, :]
            - 2.0 * jnp.einsum('bnc,bmc->bnm', src, dst))


def _query_ball(radius, nsample, xyz, new_xyz):
    b, n, _ = xyz.shape
    s = new_xyz.shape[1]
    sqrdists = _sqdist_jax(new_xyz, xyz)
    group_idx = jnp.broadcast_to(jnp.arange(n, dtype=jnp.int32), (b, s, n))
    group_idx = jnp.where(sqrdists > radius ** 2, n, group_idx)
    group_idx = jnp.sort(group_idx, axis=-1)[:, :, :nsample]
    group_first = jnp.broadcast_to(group_idx[:, :, :1], group_idx.shape)
    group_idx = jnp.where(group_idx == n, group_first, group_idx)
    return group_idx


def _index_points(points, idx):
    return jax.vmap(lambda p, i: p[i])(points, idx)


def _sa_body(nlayers, k, g_ref, f_ref, c_ref, *rest):
    ws = rest[: 2 * nlayers]
    o_ref = rest[2 * nlayers]
    ks, _ = g_ref.shape[1], None
    s = c_ref.shape[1]
    g = g_ref[0]
    c = c_ref[0]
    cexp = jnp.reshape(jnp.broadcast_to(c[None, :, :], (k, s, 3)), (k * s, 3))
    x = jnp.concatenate([g - cexp, f_ref[0]], axis=1)
    for li in range(nlayers):
        w = ws[2 * li][...]
        bv = ws[2 * li + 1][...]
        x = jnp.maximum(jnp.dot(x, w) + bv, 0.0)
    acc = x[0:s]
    for kk in range(1, k):
        acc = jnp.maximum(acc, x[kk * s:(kk + 1) * s])
    o_ref[0] = acc


def _run_sa(grouped_xyz_km, grouped_feat_km, new_xyz, layers, k):
    b, ks_, _ = grouped_xyz_km.shape
    s = new_xyz.shape[1]
    c = grouped_feat_km.shape[2]
    dout = layers[-1][0].shape[1]
    nlayers = len(layers)
    args = []
    in_specs = [
        pl.BlockSpec((1, ks_, 3), lambda bb: (bb, 0, 0)),
        pl.BlockSpec((1, ks_, c), lambda bb: (bb, 0, 0)),
        pl.BlockSpec((1, s, 3), lambda bb: (bb, 0, 0)),
    ]
    for w, bv in layers:
        args.append(w)
        args.append(bv.reshape(1, -1))
        in_specs.append(pl.BlockSpec(w.shape, lambda bb: (0, 0)))
        in_specs.append(pl.BlockSpec((1, bv.shape[0]), lambda bb: (0, 0)))
    out = pl.pallas_call(
        functools.partial(_sa_body, nlayers, k),
        grid=(b,),
        in_specs=in_specs,
        out_specs=pl.BlockSpec((1, s, dout), lambda bb: (bb, 0, 0)),
        out_shape=jax.ShapeDtypeStruct((b, s, dout), jnp.float32),
    )(grouped_xyz_km, grouped_feat_km, new_xyz, *args)
    return out


def _fp_body(nlayers, with_fc, p1_ref, x1_ref, x2t_ref, p2_ref, *rest):
    o_ref = rest[-1]
    ws = rest[:-1]
    r = x1_ref.shape[1]
    m = x2t_ref.shape[2]
    x1 = x1_ref[0]
    x2t = x2t_ref[0]
    n1 = jnp.sum(x1 * x1, axis=1, keepdims=True)
    n2 = jnp.sum(x2t * x2t, axis=0, keepdims=True)
    d = (n1 + n2) - 2.0 * jnp.dot(x1, x2t)
    lane = jax.lax.broadcasted_iota(jnp.int32, (r, m), 1)
    dd = d
    oh = jnp.zeros((r, m), jnp.float32)
    norm = jnp.zeros((r, 1), jnp.float32)
    recips = []
    idxs = []
    for _ in range(3):
        mn = jnp.min(dd, axis=1, keepdims=True)
        am = jnp.min(jnp.where(dd == mn, lane, m), axis=1, keepdims=True)
        recip = 1.0 / (jnp.maximum(mn, 0.0) + 1e-8)
        norm = norm + recip
        recips.append(recip)
        idxs.append(am)
        dd = jnp.where(lane == am, jnp.float32(jnp.inf), dd)
    for am, recip in zip(idxs, recips):
        oh = oh + jnp.where(lane == am, recip / norm, 0.0)
    interp = jnp.dot(oh, p2_ref[0])
    x = jnp.concatenate([p1_ref[0], interp], axis=1)
    for li in range(nlayers):
        w = ws[2 * li][...]
        bv = ws[2 * li + 1][...]
        x = jnp.maximum(jnp.dot(x, w) + bv, 0.0)
    if with_fc:
        w1 = ws[2 * nlayers][...]
        b1 = ws[2 * nlayers + 1][...]
        w2 = ws[2 * nlayers + 2][...]
        b2 = ws[2 * nlayers + 3][...]
        x = jnp.maximum(jnp.dot(x, w1) + b1, 0.0)
        x = jnp.dot(x, w2) + b2
    o_ref[0] = x


def _run_fp(xyz1, xyz2, points1, points2, layers, fc=None):
    b, r, c1 = points1.shape
    m, c2 = points2.shape[1], points2.shape[2]
    x2t = jnp.transpose(xyz2, (0, 2, 1))
    nlayers = len(layers)
    args = []
    in_specs = [
        pl.BlockSpec((1, r, c1), lambda bb: (bb, 0, 0)),
        pl.BlockSpec((1, r, 3), lambda bb: (bb, 0, 0)),
        pl.BlockSpec((1, 3, m), lambda bb: (bb, 0, 0)),
        pl.BlockSpec((1, m, c2), lambda bb: (bb, 0, 0)),
    ]
    all_layers = list(layers)
    if fc is not None:
        (w1, b1), (w2, b2) = fc
        w2p = jnp.pad(w2, ((0, 0), (0, 128 - w2.shape[1])))
        b2p = jnp.pad(b2, (0, 128 - b2.shape[0]))
        all_layers = all_layers + [(w1, b1), (w2p, b2p)]
    for w, bv in all_layers:
        args.append(w)
        args.append(bv.reshape(1, -1))
        in_specs.append(pl.BlockSpec(w.shape, lambda bb: (0, 0)))
        in_specs.append(pl.BlockSpec((1, bv.shape[0]), lambda bb: (0, 0)))
    dout = all_layers[-1][0].shape[1]
    out = pl.pallas_call(
        functools.partial(_fp_body, nlayers, fc is not None),
        grid=(b,),
        in_specs=in_specs,
        out_specs=pl.BlockSpec((1, r, dout), lambda bb: (bb, 0, 0)),
        out_shape=jax.ShapeDtypeStruct((b, r, dout), jnp.float32),
    )(points1, xyz1, x2t, points2, *args)
    return out


def _group_km(points, idx):
    g = _index_points(points, idx)
    g = jnp.transpose(g, (0, 2, 1, 3))
    return jnp.reshape(g, (g.shape[0], -1, g.shape[3]))


def kernel(xyz, params):
    features = xyz
    new_xyz1 = _fps_coords(xyz, 512)
    idx1 = _query_ball(0.2, 32, xyz, new_xyz1)
    g_xyz1 = _group_km(xyz, idx1)
    f1 = _run_sa(g_xyz1, g_xyz1, new_xyz1, params['sa1'], 32)
    new_xyz2 = _fps_coords(new_xyz1, 128)
    idx2 = _query_ball(0.4, 64, new_xyz1, new_xyz2)
    g_xyz2 = _group_km(new_xyz1, idx2)
    g_f1 = _group_km(f1, idx2)
    f2 = _run_sa(g_xyz2, g_f1, new_xyz2, params['sa2'], 64)
    up1 = _run_fp(new_xyz1, new_xyz2, f1, f2, params['fp1'])
    up2 = _run_fp(xyz, new_xyz1, features, up1, params['fp2'], fc=params['fc'])
    return up2[..., :NUM_CLASSES]
